# baseline (speedup 1.0000x reference)
import jax
import jax.numpy as jnp
from jax import lax
from jax.experimental import pallas as pl
from jax.experimental.pallas import tpu as pltpu
from jax.experimental.pallas import tpu_sc as plsc

B, C, HW = 16, 64, 1024
K = 1024
BETA = 0.25
NC, NS, L = 2, 16, 16
CH = C // NC
BP = B // 2
NH = HW // 2
NCHUNK = NH // L


def _tc_argmin_body(cb_ref, z_ref, idx_ref, loss_ref):
    b = pl.program_id(0)
    cb = cb_ref[...]
    z2 = z_ref[0]
    s = lax.dot_general(cb, z2, (((1,), (0,)), ((), ())),
                        preferred_element_type=jnp.float32)
    csq = jnp.sum(cb * cb, axis=1)
    zsq = jnp.sum(z2 * z2, axis=0)
    d = (zsq[None, :] + csq[:, None]) - 2.0 * s
    m = jnp.min(d, axis=0)
    kio = lax.broadcasted_iota(jnp.int32, (K, HW), 0)
    idx = jnp.min(jnp.where(d == m[None, :], kio, K), axis=0)
    idx_ref[0, 0] = idx

    @pl.when(b == 0)
    def _init():
        loss_ref[0, 0] = 0.0

    loss_ref[0, 0] += jnp.sum(m) * ((1.0 + BETA) / (B * HW * C))


def _tc_argmin(codebook_weight, z3):
    return pl.pallas_call(
        _tc_argmin_body,
        grid=(BP,),
        in_specs=[
            pl.BlockSpec((K, C), lambda b: (0, 0)),
            pl.BlockSpec((1, C, HW), lambda b: (b, 0, 0)),
        ],
        out_specs=[
            pl.BlockSpec((1, 1, HW), lambda b: (b, 0, 0)),
            pl.BlockSpec((1, 1), lambda b: (0, 0), memory_space=pltpu.SMEM),
        ],
        out_shape=[
            jax.ShapeDtypeStruct((BP, 1, HW), jnp.int32),
            jax.ShapeDtypeStruct((1, 1), jnp.float32),
        ],
    )(codebook_weight, z3)


def _sc_gather_body(cbt_hbm, idx_hbm, zq_hbm, cb_v, idx_v, out_v):
    half = lax.axis_index("c")
    s_id = lax.axis_index("s")
    b = s_id % BP
    nseg = s_id // BP
    c0 = half * CH
    n0 = nseg * NH
    pltpu.sync_copy(cbt_hbm.at[pl.ds(c0 * K, CH * K)], cb_v)
    pltpu.sync_copy(idx_hbm.at[b, pl.ds(n0, NH)], idx_v)

    @plsc.parallel_loop(0, NCHUNK, unroll=2)
    def chunk(j):
        j16 = j * L
        base = idx_v[pl.ds(j16, L)]
        vals = [plsc.load_gather(cb_v, [base + cl * K]) for cl in range(CH)]
        for cl in range(CH):
            out_v[cl, pl.ds(j16, L)] = vals[cl]

    pltpu.sync_copy(out_v, zq_hbm.at[b, pl.ds(c0, CH), pl.ds(n0, NH)])


def _sc_gather(cbt_flat, idx2):
    fn = pl.kernel(
        _sc_gather_body,
        out_type=jax.ShapeDtypeStruct((BP, C, HW), jnp.float32),
        mesh=plsc.VectorSubcoreMesh(core_axis_name="c", subcore_axis_name="s"),
        compiler_params=pltpu.CompilerParams(needs_layout_passes=False),
        scratch_types=[
            pltpu.VMEM((CH * K,), jnp.float32),
            pltpu.VMEM((NH,), jnp.int32),
            pltpu.VMEM((CH, NH), jnp.float32),
        ],
    )
    return fn(cbt_flat, idx2)


def kernel(z_e, codebook_weight):
    z3 = z_e.reshape(B, C, HW)
    cbt_flat = codebook_weight.T.reshape(C * K)
    idx3_a, loss_a = _tc_argmin(codebook_weight, z3[:BP])
    zq_a = _sc_gather(cbt_flat, idx3_a.reshape(BP, HW))
    idx3_b, loss_b = _tc_argmin(codebook_weight, z3[BP:])
    zq_b = _sc_gather(cbt_flat, idx3_b.reshape(BP, HW))
    z_q = jnp.concatenate([zq_a, zq_b], axis=0).reshape(B, C, 32, 32)
    idxc = jnp.concatenate([idx3_a, idx3_b], axis=0).reshape(B * HW, 1)
    loss = loss_a[0, 0] + loss_b[0, 0]
    return (z_q, idxc, loss)


# speedup vs baseline: 1.0511x; 1.0511x over previous
import jax
import jax.numpy as jnp
from jax import lax
from jax.experimental import pallas as pl
from jax.experimental.pallas import tpu as pltpu
from jax.experimental.pallas import tpu_sc as plsc

B, C, HW = 16, 64, 1024
K = 1024
BETA = 0.25
NC, NS, L = 2, 16, 16
CH = C // NC
BP = B // 2
NH = HW // 2
NCHUNK = NH // L


def _tc_argmin_body(cb_ref, z_ref, idx_ref, loss_ref):
    b = pl.program_id(0)
    cb = cb_ref[...]
    z2 = z_ref[0]
    s = lax.dot_general(cb, z2, (((1,), (0,)), ((), ())),
                        preferred_element_type=jnp.float32)
    csq = jnp.sum(cb * cb, axis=1)
    zsq = jnp.sum(z2 * z2, axis=0)
    d = (zsq[None, :] + csq[:, None]) - 2.0 * s
    m = jnp.min(d, axis=0)
    kio = lax.broadcasted_iota(jnp.int32, (K, HW), 0)
    idx = jnp.min(jnp.where(d == m[None, :], kio, K), axis=0)
    idx_ref[0, 0] = idx

    @pl.when(b == 0)
    def _init():
        loss_ref[0, 0] = 0.0

    loss_ref[0, 0] += jnp.sum(m) * ((1.0 + BETA) / (B * HW * C))


def _tc_argmin(codebook_weight, z3, phase):
    return pl.pallas_call(
        _tc_argmin_body,
        grid=(BP,),
        in_specs=[
            pl.BlockSpec((K, C), lambda b: (0, 0)),
            pl.BlockSpec((1, C, HW), lambda b, p=phase: (b + p * BP, 0, 0)),
        ],
        out_specs=[
            pl.BlockSpec((1, 1, HW), lambda b: (b, 0, 0)),
            pl.BlockSpec((1, 1), lambda b: (0, 0), memory_space=pltpu.SMEM),
        ],
        out_shape=[
            jax.ShapeDtypeStruct((BP, 1, HW), jnp.int32),
            jax.ShapeDtypeStruct((1, 1), jnp.float32),
        ],
    )(codebook_weight, z3)


def _sc_gather_body(cbt_hbm, idx_hbm, zq_hbm, cb_v, idx_v, out_v):
    half = lax.axis_index("c")
    s_id = lax.axis_index("s")
    b = s_id % BP
    nseg = s_id // BP
    c0 = half * CH
    n0 = nseg * NH
    pltpu.sync_copy(cbt_hbm.at[pl.ds(c0 * K, CH * K)], cb_v)
    pltpu.sync_copy(idx_hbm.at[b, pl.ds(n0, NH)], idx_v)

    @plsc.parallel_loop(0, NCHUNK, unroll=2)
    def chunk(j):
        j16 = j * L
        base = idx_v[pl.ds(j16, L)]
        vals = [plsc.load_gather(cb_v, [base + cl * K]) for cl in range(CH)]
        for cl in range(CH):
            out_v[cl, pl.ds(j16, L)] = vals[cl]

    pltpu.sync_copy(out_v, zq_hbm.at[b, pl.ds(c0, CH), pl.ds(n0, NH)])


def _sc_gather(cbt_flat, idx2):
    fn = pl.kernel(
        _sc_gather_body,
        out_type=jax.ShapeDtypeStruct((BP, C, HW), jnp.float32),
        mesh=plsc.VectorSubcoreMesh(core_axis_name="c", subcore_axis_name="s"),
        compiler_params=pltpu.CompilerParams(needs_layout_passes=False),
        scratch_types=[
            pltpu.VMEM((CH * K,), jnp.float32),
            pltpu.VMEM((NH,), jnp.int32),
            pltpu.VMEM((CH, NH), jnp.float32),
        ],
    )
    return fn(cbt_flat, idx2)


def kernel(z_e, codebook_weight):
    z3 = z_e.reshape(B, C, HW)
    cbt_flat = codebook_weight.T.reshape(C * K)
    idx3_a, loss_a = _tc_argmin(codebook_weight, z3, 0)
    zq_a = _sc_gather(cbt_flat, idx3_a.reshape(BP, HW))
    idx3_b, loss_b = _tc_argmin(codebook_weight, z3, 1)
    zq_b = _sc_gather(cbt_flat, idx3_b.reshape(BP, HW))
    z_q = jnp.concatenate([zq_a, zq_b], axis=0).reshape(B, C, 32, 32)
    idxc = jnp.concatenate([idx3_a, idx3_b], axis=0).reshape(B * HW, 1)
    loss = loss_a[0, 0] + loss_b[0, 0]
    return (z_q, idxc, loss)


# speedup vs baseline: 1.1564x; 1.1001x over previous
import jax
import jax.numpy as jnp
from jax import lax
from jax.experimental import pallas as pl
from jax.experimental.pallas import tpu as pltpu
from jax.experimental.pallas import tpu_sc as plsc

B, C, HW = 16, 64, 1024
K = 1024
BETA = 0.25
NC, NS, L = 2, 16, 16
CH = C // NC
NCHUNK = HW // L


def _tc_argmin_body(cb_ref, z_ref, idx_ref, loss_ref):
    b = pl.program_id(0)
    cb = cb_ref[...]
    z2 = z_ref[0]
    s = lax.dot_general(cb, z2, (((1,), (0,)), ((), ())),
                        preferred_element_type=jnp.float32)
    csq = jnp.sum(cb * cb, axis=1)
    zsq = jnp.sum(z2 * z2, axis=0)
    d = (zsq[None, :] + csq[:, None]) - 2.0 * s
    m = jnp.min(d, axis=0)
    idx_ref[0, 0] = jnp.argmin(d, axis=0).astype(jnp.int32)

    @pl.when(b == 0)
    def _init():
        loss_ref[0, 0] = 0.0

    loss_ref[0, 0] += jnp.sum(m) * ((1.0 + BETA) / (B * HW * C))


def _sc_gather_body(cbt_hbm, idx_hbm, zq_hbm, cb_v, idx_v, out_v):
    half = lax.axis_index("c")
    b = lax.axis_index("s")
    c0 = half * CH
    pltpu.sync_copy(cbt_hbm.at[pl.ds(c0 * K, CH * K)], cb_v)
    pltpu.sync_copy(idx_hbm.at[b], idx_v)

    @plsc.parallel_loop(0, NCHUNK, unroll=2)
    def chunk(j):
        j16 = j * L
        base = idx_v[pl.ds(j16, L)]
        vals = [plsc.load_gather(cb_v, [base + cl * K]) for cl in range(CH)]
        for cl in range(CH):
            out_v[cl, pl.ds(j16, L)] = vals[cl]

    pltpu.sync_copy(out_v, zq_hbm.at[b, pl.ds(c0, CH)])


def _sc_gather(codebook_weight, idx2):
    fn = pl.kernel(
        _sc_gather_body,
        out_type=jax.ShapeDtypeStruct((B, C, HW), jnp.float32),
        mesh=plsc.VectorSubcoreMesh(core_axis_name="c", subcore_axis_name="s"),
        compiler_params=pltpu.CompilerParams(needs_layout_passes=False),
        scratch_types=[
            pltpu.VMEM((CH * K,), jnp.float32),
            pltpu.VMEM((HW,), jnp.int32),
            pltpu.VMEM((CH, HW), jnp.float32),
        ],
    )
    return fn(codebook_weight.T.reshape(C * K), idx2)


def _tc_finalize_body(zq_ref, out_ref):
    x = zq_ref[0]
    cols = [x[:, h * 32:(h + 1) * 32].reshape(C, 1, 32) for h in range(32)]
    out_ref[0] = jnp.concatenate(cols, axis=1)


def kernel(z_e, codebook_weight):
    idx3, loss = pl.pallas_call(
        _tc_argmin_body,
        grid=(B,),
        in_specs=[
            pl.BlockSpec((K, C), lambda b: (0, 0)),
            pl.BlockSpec((1, C, HW), lambda b: (b, 0, 0)),
        ],
        out_specs=[
            pl.BlockSpec((1, 1, HW), lambda b: (b, 0, 0)),
            pl.BlockSpec((1, 1), lambda b: (0, 0), memory_space=pltpu.SMEM),
        ],
        out_shape=[
            jax.ShapeDtypeStruct((B, 1, HW), jnp.int32),
            jax.ShapeDtypeStruct((1, 1), jnp.float32),
        ],
    )(codebook_weight, z_e.reshape(B, C, HW))
    idxc = idx3.reshape(B * HW, 1)
    idx2 = idx3.reshape(B, HW)
    zq = _sc_gather(codebook_weight, idx2)
    z_q = zq.reshape(B, C, 32, 32)
    return (z_q, idxc, loss[0, 0])
